# baseline (speedup 1.0000x reference)
import functools

import jax
import jax.numpy as jnp
from jax import lax
from jax.experimental import pallas as pl
from jax.experimental.pallas import tpu as pltpu
from jax.experimental.pallas import tpu_sc as plsc

B = 64
IN_F = 16384
OUT_F = 16384
LANES = 16
E_CHUNK = 16384
UNROLL = 8

_SELU_SCALE = 1.0507009873554805
_SELU_ALPHA = 1.6732632423543772


def _selu(v):
    return _SELU_SCALE * jnp.where(
        v > 0.0, v, _SELU_ALPHA * (jnp.exp(jnp.minimum(v, 0.0)) - 1.0))


def _sc_body(feat_hbm, edges_hbm, bias_hbm, out_hbm,
             xp, y0, y1, eb0, eb1, semi, sema, semb):
    nc = 2
    wid = lax.axis_index("s") * nc + lax.axis_index("c")
    b0 = wid * 2
    n_chunks = edges_hbm.shape[0]

    cx = pltpu.async_copy(feat_hbm.at[wid], xp, semi)
    cy0 = pltpu.async_copy(bias_hbm, y0, semi)
    cy1 = pltpu.async_copy(bias_hbm, y1, semi)
    pltpu.async_copy(edges_hbm.at[0], eb0, sema)
    cx.wait()
    cy0.wait()
    cy1.wait()

    hi_mask = jnp.full((LANES,), jnp.int32(-65536), jnp.int32)
    lo14 = jnp.full((LANES,), 0x3FFF, jnp.int32)

    def process(ebuf):
        @plsc.parallel_loop(0, E_CHUNK // LANES, unroll=UNROLL)
        def _(i):
            off = i * LANES
            cr = ebuf[0, pl.ds(off, LANES)]
            w = plsc.bitcast(ebuf[1, pl.ds(off, LANES)], jnp.float32)
            c = cr & lo14
            r = lax.shift_right_logical(cr, 14)
            g = plsc.load_gather(xp, [c])
            g0 = plsc.bitcast(g & hi_mask, jnp.float32)
            g1 = plsc.bitcast(g << 16, jnp.float32)
            plsc.addupdate_scatter(y0, [r], g0 * w)
            plsc.addupdate_scatter(y1, [r], g1 * w)

    def pair_body(k, carry):
        c0 = 2 * k
        pltpu.async_copy(edges_hbm.at[c0 + 1], eb1, semb)
        pltpu.make_async_copy(edges_hbm.at[c0], eb0, sema).wait()
        process(eb0)

        @pl.when(c0 + 2 < n_chunks)
        def _():
            pltpu.async_copy(edges_hbm.at[c0 + 2], eb0, sema)

        pltpu.make_async_copy(edges_hbm.at[c0 + 1], eb1, semb).wait()
        process(eb1)
        return carry

    lax.fori_loop(0, n_chunks // 2, pair_body, 0)

    @plsc.parallel_loop(0, OUT_F // LANES, unroll=8)
    def _(i):
        off = i * LANES
        y0[pl.ds(off, LANES)] = _selu(y0[pl.ds(off, LANES)])
        y1[pl.ds(off, LANES)] = _selu(y1[pl.ds(off, LANES)])

    pltpu.sync_copy(y0, out_hbm.at[b0])
    pltpu.sync_copy(y1, out_hbm.at[b0 + 1])


@jax.jit
def _run(features, edges, bias):
    mesh = plsc.VectorSubcoreMesh(core_axis_name="c", subcore_axis_name="s")
    f = functools.partial(
        pl.kernel,
        mesh=mesh,
        out_type=jax.ShapeDtypeStruct((B, OUT_F), jnp.float32),
        compiler_params=pltpu.CompilerParams(needs_layout_passes=False),
        scratch_types=[
            pltpu.VMEM((IN_F,), jnp.int32),
            pltpu.VMEM((OUT_F,), jnp.float32),
            pltpu.VMEM((OUT_F,), jnp.float32),
            pltpu.VMEM((2, E_CHUNK), jnp.int32),
            pltpu.VMEM((2, E_CHUNK), jnp.int32),
            pltpu.SemaphoreType.DMA,
            pltpu.SemaphoreType.DMA,
            pltpu.SemaphoreType.DMA,
        ],
    )(_sc_body)
    return f(features, edges, bias)


def kernel(features, rows, cols, w_vals, bias):
    nnz = rows.shape[0]
    n_chunks = -(-nnz // E_CHUNK)
    n_chunks += n_chunks % 2
    n_pad = n_chunks * E_CHUNK - nnz
    rows_p = jnp.pad(rows.astype(jnp.int32), (0, n_pad))
    cols_p = jnp.pad(cols.astype(jnp.int32), (0, n_pad))
    w_bits = jnp.pad(lax.bitcast_convert_type(w_vals, jnp.int32), (0, n_pad))
    cr = cols_p | (rows_p << 14)
    edges = jnp.stack([cr, w_bits], axis=0)
    edges = edges.reshape(2, n_chunks, E_CHUNK).transpose(1, 0, 2)
    fb = lax.bitcast_convert_type(
        features.astype(jnp.bfloat16), jnp.uint16).astype(jnp.uint32)
    feat_packed = lax.bitcast_convert_type(
        (fb[0::2, :] << 16) | fb[1::2, :], jnp.int32)
    return _run(feat_packed, edges, bias)


# speedup vs baseline: 1.2416x; 1.2416x over previous
import functools

import jax
import jax.numpy as jnp
from jax import lax
from jax.experimental import pallas as pl
from jax.experimental.pallas import tpu as pltpu
from jax.experimental.pallas import tpu_sc as plsc

B = 64
IN_F = 16384
OUT_F = 16384
LANES = 16
E_CHUNK = 8192
UNROLL = 16

_SELU_SCALE = 1.0507009873554805
_SELU_ALPHA = 1.6732632423543772


def _selu(v):
    return _SELU_SCALE * jnp.where(
        v > 0.0, v, _SELU_ALPHA * (jnp.exp(jnp.minimum(v, 0.0)) - 1.0))


def _sc_body(feat_hbm, edges_hbm, bias_hbm, out_hbm,
             xp, y0, y1, eb0, eb1, semi, sema, semb):
    nc = 2
    wid = lax.axis_index("s") * nc + lax.axis_index("c")
    b0 = wid * 2
    n_chunks = edges_hbm.shape[0]

    cx = pltpu.async_copy(feat_hbm.at[wid], xp, semi)
    cy0 = pltpu.async_copy(bias_hbm, y0, semi)
    cy1 = pltpu.async_copy(bias_hbm, y1, semi)
    pltpu.async_copy(edges_hbm.at[0], eb0, sema)
    cx.wait()
    cy0.wait()
    cy1.wait()

    hi_mask = jnp.full((LANES,), jnp.int32(-65536), jnp.int32)
    lo14 = jnp.full((LANES,), 0x3FFF, jnp.int32)

    def process(ebuf):
        @plsc.parallel_loop(0, E_CHUNK // LANES, unroll=UNROLL)
        def _(i):
            off = i * LANES
            cr = ebuf[0, pl.ds(off, LANES)]
            w = plsc.bitcast(ebuf[1, pl.ds(off, LANES)], jnp.float32)
            c = cr & lo14
            r = lax.shift_right_logical(cr, 14)
            g = plsc.load_gather(xp, [c])
            g0 = plsc.bitcast(g & hi_mask, jnp.float32)
            g1 = plsc.bitcast(g << 16, jnp.float32)
            plsc.addupdate_scatter(y0, [r], g0 * w)
            plsc.addupdate_scatter(y1, [r], g1 * w)

    def pair_body(k, carry):
        c0 = 2 * k
        pltpu.async_copy(edges_hbm.at[c0 + 1], eb1, semb)
        pltpu.make_async_copy(edges_hbm.at[c0], eb0, sema).wait()
        process(eb0)

        @pl.when(c0 + 2 < n_chunks)
        def _():
            pltpu.async_copy(edges_hbm.at[c0 + 2], eb0, sema)

        pltpu.make_async_copy(edges_hbm.at[c0 + 1], eb1, semb).wait()
        process(eb1)
        return carry

    lax.fori_loop(0, n_chunks // 2, pair_body, 0)

    @plsc.parallel_loop(0, OUT_F // LANES, unroll=8)
    def _(i):
        off = i * LANES
        y0[pl.ds(off, LANES)] = _selu(y0[pl.ds(off, LANES)])
        y1[pl.ds(off, LANES)] = _selu(y1[pl.ds(off, LANES)])

    pltpu.sync_copy(y0, out_hbm.at[b0])
    pltpu.sync_copy(y1, out_hbm.at[b0 + 1])


@jax.jit
def _run(features, edges, bias):
    mesh = plsc.VectorSubcoreMesh(core_axis_name="c", subcore_axis_name="s")
    f = functools.partial(
        pl.kernel,
        mesh=mesh,
        out_type=jax.ShapeDtypeStruct((B, OUT_F), jnp.float32),
        compiler_params=pltpu.CompilerParams(needs_layout_passes=False),
        scratch_types=[
            pltpu.VMEM((IN_F,), jnp.int32),
            pltpu.VMEM((OUT_F,), jnp.float32),
            pltpu.VMEM((OUT_F,), jnp.float32),
            pltpu.VMEM((2, E_CHUNK), jnp.int32),
            pltpu.VMEM((2, E_CHUNK), jnp.int32),
            pltpu.SemaphoreType.DMA,
            pltpu.SemaphoreType.DMA,
            pltpu.SemaphoreType.DMA,
        ],
    )(_sc_body)
    return f(features, edges, bias)


def kernel(features, rows, cols, w_vals, bias):
    nnz = rows.shape[0]
    n_chunks = -(-nnz // E_CHUNK)
    n_chunks += n_chunks % 2
    n_pad = n_chunks * E_CHUNK - nnz
    rows_p = jnp.pad(rows.astype(jnp.int32), (0, n_pad))
    cols_p = jnp.pad(cols.astype(jnp.int32), (0, n_pad))
    w_bits = jnp.pad(lax.bitcast_convert_type(w_vals, jnp.int32), (0, n_pad))
    cr = cols_p | (rows_p << 14)
    edges = jnp.stack([cr, w_bits], axis=0)
    edges = edges.reshape(2, n_chunks, E_CHUNK).transpose(1, 0, 2)
    fb = lax.bitcast_convert_type(
        features.astype(jnp.bfloat16), jnp.uint16).astype(jnp.uint32)
    feat_packed = lax.bitcast_convert_type(
        (fb[0::2, :] << 16) | fb[1::2, :], jnp.int32)
    return _run(feat_packed, edges, bias)
